# baseline (speedup 1.0000x reference)
import jax
import jax.numpy as jnp
from jax.experimental import pallas as pl
from jax.experimental.pallas import tpu as pltpu

_N = 10000
_F = 128
_BM_SUP = 2000
_BM = 200


def _support_kernel(x_ref, w_ref, b_ref, sup32_ref, sup16_ref):
    s = jnp.dot(x_ref[...], w_ref[...], preferred_element_type=jnp.float32)
    s = s + b_ref[...]
    sup32_ref[...] = s
    sup16_ref[...] = s.astype(jnp.bfloat16)


def _agg_kernel(adj_ref, sup16_ref, sup32_ref, dn_ref, out_ref):
    neigh = jnp.dot(adj_ref[...].astype(jnp.bfloat16), sup16_ref[...],
                    preferred_element_type=jnp.float32)
    out_ref[...] = jnp.maximum(sup32_ref[...] - dn_ref[...] * neigh, 0.0)


def kernel(x, adj_matrix, degree_norm, W, b):
    b2 = b.reshape(1, _F)
    sup32, sup16 = pl.pallas_call(
        _support_kernel,
        grid=(_N // _BM_SUP,),
        in_specs=[
            pl.BlockSpec((_BM_SUP, _F), lambda i: (i, 0)),
            pl.BlockSpec((_F, _F), lambda i: (0, 0)),
            pl.BlockSpec((1, _F), lambda i: (0, 0)),
        ],
        out_specs=[
            pl.BlockSpec((_BM_SUP, _F), lambda i: (i, 0)),
            pl.BlockSpec((_BM_SUP, _F), lambda i: (i, 0)),
        ],
        out_shape=[
            jax.ShapeDtypeStruct((_N, _F), jnp.float32),
            jax.ShapeDtypeStruct((_N, _F), jnp.bfloat16),
        ],
        compiler_params=pltpu.CompilerParams(
            dimension_semantics=("arbitrary",)),
    )(x, W, b2)

    out = pl.pallas_call(
        _agg_kernel,
        grid=(_N // _BM,),
        in_specs=[
            pl.BlockSpec((_BM, _N), lambda i: (i, 0)),
            pl.BlockSpec((_N, _F), lambda i: (0, 0)),
            pl.BlockSpec((_BM, _F), lambda i: (i, 0)),
            pl.BlockSpec((_BM, 1), lambda i: (i, 0)),
        ],
        out_specs=pl.BlockSpec((_BM, _F), lambda i: (i, 0)),
        out_shape=jax.ShapeDtypeStruct((_N, _F), jnp.float32),
        compiler_params=pltpu.CompilerParams(
            dimension_semantics=("parallel",)),
    )(adj_matrix, sup16, sup32, degree_norm)
    return out
